# Initial kernel scaffold; baseline (speedup 1.0000x reference)
import jax
import jax.numpy as jnp
from jax.experimental import pallas as pl


def kernel(x, weight_norm, bias, scale):
    raise NotImplementedError("write your pallas kernel here")


# speedup vs baseline: 1.8898x; 1.8898x over previous
import jax
import jax.numpy as jnp
from jax.experimental import pallas as pl
from jax.experimental.pallas import tpu as pltpu


def _conv3x3_kernel(x_ref, w_ref, b_ref, o_ref, p_ref):
    hp, wp, c_in = x_ref.shape
    h, w = hp - 2, wp - 2
    for ky in range(3):
        for kx in range(3):
            t = ky * 3 + kx
            p_ref[:, t * c_in:(t + 1) * c_in] = (
                x_ref[ky:ky + h, kx:kx + w, :].reshape(h * w, c_in))
    acc = jax.lax.dot_general(
        w_ref[...], p_ref[...],
        dimension_numbers=(((0,), (1,)), ((), ())),
        preferred_element_type=jnp.float32,
    )
    o_ref[...] = acc + b_ref[...]


def kernel(x, weight_norm, bias, scale):
    b, c_in, h, w = x.shape
    c_out, _, k_size, _ = weight_norm.shape
    k_dim = k_size * k_size * c_in

    w_mat = (weight_norm * jnp.asarray(scale, weight_norm.dtype)
             ).transpose(2, 3, 1, 0).reshape(k_dim, c_out).astype(jnp.bfloat16)

    x_nhwc = jnp.transpose(x, (0, 2, 3, 1)).astype(jnp.bfloat16)
    x_pad = jnp.pad(x_nhwc, ((0, 0), (1, 1), (1, 1), (0, 0)))

    bias_col = bias.astype(jnp.float32).reshape(c_out, 1)

    out = pl.pallas_call(
        _conv3x3_kernel,
        out_shape=jax.ShapeDtypeStruct((b, c_out, h * w), jnp.float32),
        grid=(b,),
        in_specs=[
            pl.BlockSpec((None, h + 2, w + 2, c_in), lambda i: (i, 0, 0, 0)),
            pl.BlockSpec((k_dim, c_out), lambda i: (0, 0)),
            pl.BlockSpec((c_out, 1), lambda i: (0, 0)),
        ],
        out_specs=pl.BlockSpec((None, c_out, h * w), lambda i: (i, 0, 0)),
        scratch_shapes=[pltpu.VMEM((h * w, k_dim), jnp.bfloat16)],
        compiler_params=pltpu.CompilerParams(
            dimension_semantics=("parallel",),
            vmem_limit_bytes=64 * 1024 * 1024,
        ),
    )(x_pad, w_mat, bias_col)

    return out.reshape(b, c_out, h, w).astype(x.dtype)
